# Initial kernel scaffold; baseline (speedup 1.0000x reference)
import jax
import jax.numpy as jnp
from jax.experimental import pallas as pl


def kernel(values, indices, w, v, b):
    raise NotImplementedError("write your pallas kernel here")


# speedup vs baseline: 1.3980x; 1.3980x over previous
import jax
import jax.numpy as jnp
from jax import lax
from jax.experimental import pallas as pl
from jax.experimental.pallas import tpu as pltpu
from jax.experimental.pallas import tpu_sc as plsc

B = 16384
F = 26
K = 16
NC, NS = 2, 16
NW = NC * NS
RPW = B // NW
EPW = RPW * F
BLK_ROWS = 64
NBLK = RPW // BLK_ROWS
BLK_E = BLK_ROWS * F
CH = 128
NCH = BLK_E // CH
GROUPS = RPW // K


def _fm_body(idx_hbm, val_hbm, w_hbm, v_hbm, b_hbm, out_hbm,
             idxv, valv, wbuf, vbuf0, vbuf1, dbuf, outv, bv,
             sem_v0, sem_v1, sem_w):
    wid = lax.axis_index("s") * NC + lax.axis_index("c")
    ebase = wid * EPW
    rbase = wid * RPW

    pltpu.sync_copy(idx_hbm.at[pl.ds(ebase, EPW)], idxv)
    pltpu.sync_copy(val_hbm.at[pl.ds(ebase, EPW)], valv)
    pltpu.sync_copy(b_hbm, bv)

    vbufs = (vbuf0, vbuf1)
    sems = (sem_v0, sem_v1)
    iota = lax.iota(jnp.int32, K)

    def fire_v(blk):
        buf, sem = vbufs[blk % 2], sems[blk % 2]
        hs = []
        for c in range(NCH):
            off = blk * BLK_E + c * CH
            hs.append(pltpu.async_copy(
                v_hbm.at[idxv.at[pl.ds(off, CH)]],
                buf.at[pl.ds(c * CH, CH)], sem))
        return hs

    def fire_w(blk):
        hs = []
        for c in range(NCH):
            off = blk * BLK_E + c * CH
            hs.append(pltpu.async_copy(
                w_hbm.at[idxv.at[pl.ds(off, CH)]],
                wbuf.at[pl.ds(off, CH)], sem_w))
        return hs

    hv = fire_v(0)
    w_hs = fire_w(0)

    for blk in range(NBLK):
        hv_next = None
        if blk + 1 < NBLK:
            hv_next = fire_v(blk + 1)
            w_hs += fire_w(blk + 1)
        for h in hv:
            h.wait()
        buf = vbufs[blk % 2]

        def row_body(r, carry, blk=blk, buf=buf):
            e0 = blk * BLK_E + r * F
            va = valv[pl.ds(e0, K)]
            vb = valv[pl.ds(e0 + (F - K), K)]
            acc = jnp.zeros((K,), jnp.float32)
            acc2 = jnp.zeros((K,), jnp.float32)
            for f in range(F):
                x = buf[r * F + f, :]
                val = va[f] if f < K else vb[f - (F - K)]
                xe = x * val
                acc = acc + xe
                acc2 = acc2 + xe * xe
            d = acc * acc - acc2
            plsc.store_scatter(dbuf, [iota * RPW + (blk * BLK_ROWS + r)], d)
            return carry

        lax.fori_loop(0, BLK_ROWS, row_body, 0)
        hv = hv_next

    for h in w_hs:
        h.wait()
    bvec = bv[...]

    def grp_body(g, carry):
        pacc = jnp.zeros((K,), jnp.float32)
        for k in range(K):
            pacc = pacc + dbuf[pl.ds(k * RPW + g * K, K)]
        wacc = jnp.zeros((K,), jnp.float32)
        eidx0 = iota * F + g * (K * F)
        for f in range(F):
            eidx = eidx0 + f
            wacc = wacc + (plsc.load_gather(wbuf, [eidx]) *
                           plsc.load_gather(valv, [eidx]))
        logit = wacc + bvec + 0.5 * pacc
        y = 1.0 / (1.0 + jnp.exp(-logit))
        outv[pl.ds(g * K, K)] = y
        return carry

    lax.fori_loop(0, GROUPS, grp_body, 0)
    pltpu.sync_copy(outv, out_hbm.at[pl.ds(rbase, RPW)])


def kernel(values, indices, w, v, b):
    idx_flat = indices.reshape(-1)
    val_flat = values.reshape(-1)
    w_flat = w.reshape(-1)
    b16 = jnp.broadcast_to(b, (K,))
    mesh = plsc.VectorSubcoreMesh(core_axis_name="c", subcore_axis_name="s",
                                  num_cores=NC, num_subcores=NS)
    fm = pl.kernel(
        _fm_body,
        out_type=jax.ShapeDtypeStruct((B,), jnp.float32),
        mesh=mesh,
        compiler_params=pltpu.CompilerParams(needs_layout_passes=False,
                                             use_tc_tiling_on_sc=False),
        scratch_types=[
            pltpu.VMEM((EPW,), jnp.int32),
            pltpu.VMEM((EPW,), jnp.float32),
            pltpu.VMEM((EPW,), jnp.float32),
            pltpu.VMEM((BLK_E, K), jnp.float32),
            pltpu.VMEM((BLK_E, K), jnp.float32),
            pltpu.VMEM((K * RPW,), jnp.float32),
            pltpu.VMEM((RPW,), jnp.float32),
            pltpu.VMEM((K,), jnp.float32),
            pltpu.SemaphoreType.DMA,
            pltpu.SemaphoreType.DMA,
            pltpu.SemaphoreType.DMA,
        ],
    )
    return fm(idx_flat, val_flat, w_flat, v, b16)


# speedup vs baseline: 1.4379x; 1.0286x over previous
import jax
import jax.numpy as jnp
from jax import lax
from jax.experimental import pallas as pl
from jax.experimental.pallas import tpu as pltpu
from jax.experimental.pallas import tpu_sc as plsc

V = 1000000
B = 16384
F = 26
K = 16
NC, NS = 2, 16
NW = NC * NS
RPW = B // NW
EPW = RPW * F
BLK_ROWS = 64
NBLK = RPW // BLK_ROWS
BLK_E = BLK_ROWS * F
CH = 128
NCH = BLK_E // CH
GROUPS = RPW // K

TC_COLS = 2048
TC_GRID = -(-V // TC_COLS)
V8 = V // 8


def _relayout_body(vt_ref, out_ref):
    x = vt_ref[...]
    k_ids = lax.broadcasted_iota(jnp.int32, (K, 128), 0)
    m_ids = lax.broadcasted_iota(jnp.int32, (K, 128), 1)
    sel_k = jnp.where(m_ids % K == k_ids, 1.0, 0.0)
    t0 = lax.dot_general(x, sel_k, (((0,), (0,)), ((), ())),
                         preferred_element_type=jnp.float32)
    s_ids = lax.broadcasted_iota(jnp.int32, (8, 128), 0)
    m2_ids = lax.broadcasted_iota(jnp.int32, (8, 128), 1)
    sel_s = jnp.where(m2_ids // K == s_ids, 1.0, 0.0)
    t1 = t0.reshape(TC_COLS // 8, 8, 128) * sel_s[None]
    out_ref[...] = jnp.sum(t1, axis=1)


def _fm_body(idx_hbm, val_hbm, w_hbm, v128_hbm, b_hbm, out_hbm,
             idxv, valv, wbuf, vbuf0, vbuf1, dbuf, outv, bv,
             sem_v0, sem_v1, sem_w):
    wid = lax.axis_index("s") * NC + lax.axis_index("c")
    ebase = wid * EPW
    rbase = wid * RPW
    vtab = v128_hbm

    pltpu.sync_copy(idx_hbm.at[pl.ds(ebase, EPW)], idxv)
    pltpu.sync_copy(val_hbm.at[pl.ds(ebase, EPW)], valv)
    pltpu.sync_copy(b_hbm, bv)

    vbufs = (vbuf0, vbuf1)
    sems = (sem_v0, sem_v1)
    iota = lax.iota(jnp.int32, K)

    def fire_v(blk):
        buf, sem = vbufs[blk % 2], sems[blk % 2]
        hs = []
        for c in range(NCH):
            off = blk * BLK_E + c * CH
            hs.append(pltpu.async_copy(
                vtab.at[idxv.at[pl.ds(off, CH)]],
                buf.at[pl.ds(c * CH, CH)], sem))
        return hs

    def fire_w(blk):
        hs = []
        for c in range(NCH):
            off = blk * BLK_E + c * CH
            hs.append(pltpu.async_copy(
                w_hbm.at[idxv.at[pl.ds(off, CH)]],
                wbuf.at[pl.ds(off, CH)], sem_w))
        return hs

    hv = fire_v(0)
    w_hs = fire_w(0)

    for blk in range(NBLK):
        hv_next = None
        if blk + 1 < NBLK:
            hv_next = fire_v(blk + 1)
            w_hs += fire_w(blk + 1)
        for h in hv:
            h.wait()
        buf = vbufs[blk % 2]

        def row_body(r, carry, blk=blk, buf=buf):
            e0 = blk * BLK_E + r * F
            va = valv[pl.ds(e0, K)]
            vb = valv[pl.ds(e0 + (F - K), K)]
            acc = jnp.zeros((K,), jnp.float32)
            acc2 = jnp.zeros((K,), jnp.float32)
            for f in range(F):
                x = buf[r * F + f, :]
                val = va[f] if f < K else vb[f - (F - K)]
                xe = x * val
                acc = acc + xe
                acc2 = acc2 + xe * xe
            d = acc * acc - acc2
            plsc.store_scatter(dbuf, [iota * RPW + (blk * BLK_ROWS + r)], d)
            return carry

        lax.fori_loop(0, BLK_ROWS, row_body, 0)
        hv = hv_next

    for h in w_hs:
        h.wait()
    bvec = bv[...]

    def grp_body(g, carry):
        pacc = jnp.zeros((K,), jnp.float32)
        for k in range(K):
            pacc = pacc + dbuf[pl.ds(k * RPW + g * K, K)]
        wacc = jnp.zeros((K,), jnp.float32)
        eidx0 = iota * F + g * (K * F)
        for f in range(F):
            eidx = eidx0 + f
            wacc = wacc + (plsc.load_gather(wbuf, [eidx]) *
                           plsc.load_gather(valv, [eidx]))
        logit = wacc + bvec + 0.5 * pacc
        y = 1.0 / (1.0 + jnp.exp(-logit))
        outv[pl.ds(g * K, K)] = y
        return carry

    lax.fori_loop(0, GROUPS, grp_body, 0)
    pltpu.sync_copy(outv, out_hbm.at[pl.ds(rbase, RPW)])


def kernel(values, indices, w, v, b):
    v128 = pl.pallas_call(
        _relayout_body,
        grid=(TC_GRID,),
        in_specs=[pl.BlockSpec((K, TC_COLS), lambda i: (0, i))],
        out_specs=pl.BlockSpec((TC_COLS // 8, 128), lambda i: (i, 0)),
        out_shape=jax.ShapeDtypeStruct((V8, 128), jnp.float32),
    )(v.T)
    v16 = v128.reshape(V, K)

    idx_flat = indices.reshape(-1)
    val_flat = values.reshape(-1)
    w_flat = w.reshape(-1)
    b16 = jnp.broadcast_to(b, (K,))
    mesh = plsc.VectorSubcoreMesh(core_axis_name="c", subcore_axis_name="s",
                                  num_cores=NC, num_subcores=NS)
    fm = pl.kernel(
        _fm_body,
        out_type=jax.ShapeDtypeStruct((B,), jnp.float32),
        mesh=mesh,
        compiler_params=pltpu.CompilerParams(needs_layout_passes=False,
                                             use_tc_tiling_on_sc=False),
        scratch_types=[
            pltpu.VMEM((EPW,), jnp.int32),
            pltpu.VMEM((EPW,), jnp.float32),
            pltpu.VMEM((EPW,), jnp.float32),
            pltpu.VMEM((BLK_E, K), jnp.float32),
            pltpu.VMEM((BLK_E, K), jnp.float32),
            pltpu.VMEM((K * RPW,), jnp.float32),
            pltpu.VMEM((RPW,), jnp.float32),
            pltpu.VMEM((K,), jnp.float32),
            pltpu.SemaphoreType.DMA,
            pltpu.SemaphoreType.DMA,
            pltpu.SemaphoreType.DMA,
        ],
    )
    return fm(idx_flat, val_flat, w_flat, v16, b16)
